# baseline (speedup 1.0000x reference)
import functools

import jax
import jax.numpy as jnp
from jax import lax
from jax.experimental import pallas as pl
from jax.experimental.pallas import tpu as pltpu
from jax.experimental.pallas import tpu_sc as plsc

_NUM_SEGMENTS = 2
_EMBED_DIM = 1024
_BATCH = 4
_SEQ = 8192
_TOKENS = _BATCH * _SEQ
_NW = 32
_TOK_PER_W = _TOKENS // _NW
_CHUNK = 32
_NCHUNK = _TOK_PER_W // _CHUNK

_mesh = plsc.VectorSubcoreMesh(core_axis_name="c", subcore_axis_name="s")


@functools.partial(
    pl.kernel,
    mesh=_mesh,
    out_type=jax.ShapeDtypeStruct((_TOKENS, _EMBED_DIM), jnp.float32),
    scratch_types=[
        pltpu.VMEM((_TOK_PER_W,), jnp.int32),
        pltpu.VMEM((_CHUNK, _EMBED_DIM), jnp.float32),
        pltpu.VMEM((_CHUNK, _EMBED_DIM), jnp.float32),
        pltpu.SemaphoreType.DMA,
        pltpu.SemaphoreType.DMA,
    ],
)
def _segment_gather(idx_hbm, table_hbm, out_hbm, idx_v, rows0, rows1, sem0, sem1):
    wid = lax.axis_index("s") * 2 + lax.axis_index("c")
    base = wid * _TOK_PER_W
    pltpu.sync_copy(idx_hbm.at[pl.ds(base, _TOK_PER_W)], idx_v)
    bufs = (rows0, rows1)
    sems = (sem0, sem1)

    def gather(i):
        ichunk = idx_v.at[pl.ds(i * _CHUNK, _CHUNK)]
        return pltpu.async_copy(table_hbm.at[ichunk], bufs[i % 2], sems[i % 2])

    pending = gather(0)
    for i in range(_NCHUNK):
        nxt = gather(i + 1) if i + 1 < _NCHUNK else None
        pending.wait()
        pltpu.sync_copy(bufs[i % 2], out_hbm.at[pl.ds(base + i * _CHUNK, _CHUNK)])
        pending = nxt


def kernel(inputs, segment_embed_weights):
    idx = inputs.astype(jnp.int32).reshape(_TOKENS)
    out = _segment_gather(idx, segment_embed_weights)
    return (out.reshape(_BATCH, _SEQ, _EMBED_DIM), segment_embed_weights)


# speedup vs baseline: 13.7483x; 13.7483x over previous
import functools

import jax
import jax.numpy as jnp
from jax import lax
from jax.experimental import pallas as pl
from jax.experimental.pallas import tpu as pltpu
from jax.experimental.pallas import tpu_sc as plsc

_NUM_SEGMENTS = 2
_EMBED_DIM = 1024
_BATCH = 4
_SEQ = 8192
_TOKENS = _BATCH * _SEQ
_NW = 32
_TOK_PER_W = _TOKENS // _NW
_DRAIN_ROWS = 16

_mesh = plsc.VectorSubcoreMesh(core_axis_name="c", subcore_axis_name="s")


@functools.partial(
    pl.kernel,
    mesh=_mesh,
    out_type=jax.ShapeDtypeStruct((_TOKENS, _EMBED_DIM), jnp.float32),
    scratch_types=[
        pltpu.SMEM((_TOK_PER_W,), jnp.int32),
        pltpu.VMEM((_NUM_SEGMENTS, _EMBED_DIM), jnp.float32),
        pltpu.VMEM((_DRAIN_ROWS, _EMBED_DIM), jnp.float32),
        pltpu.VMEM_SHARED((_TOKENS,), jnp.int32),
        pltpu.SemaphoreType.DMA,
    ],
)
def _segment_gather(idx_hbm, table_hbm, out_hbm, idx_s, table_v, drain_v,
                    idx_sp, sem):
    sid = lax.axis_index("s")
    wid = sid * 2 + lax.axis_index("c")
    base = wid * _TOK_PER_W
    pltpu.sync_copy(table_hbm, table_v)

    @pl.when(sid == 0)
    def _():
        pltpu.sync_copy(idx_hbm, idx_sp)

    plsc.subcore_barrier()
    pltpu.sync_copy(idx_sp.at[pl.ds(base, _TOK_PER_W)], idx_s)

    def body(t, carry):
        s = idx_s[t]
        pltpu.async_copy(table_v.at[s], out_hbm.at[base + t], sem)
        return carry

    lax.fori_loop(0, _TOK_PER_W, body, 0)

    def dbody(i, carry):
        pltpu.make_async_copy(out_hbm.at[pl.ds(base, _DRAIN_ROWS)], drain_v,
                              sem).wait()
        return carry

    lax.fori_loop(0, _TOK_PER_W // _DRAIN_ROWS, dbody, 0)


def kernel(inputs, segment_embed_weights):
    idx = inputs.astype(jnp.int32).reshape(_TOKENS)
    out = _segment_gather(idx, segment_embed_weights)
    return (out.reshape(_BATCH, _SEQ, _EMBED_DIM), segment_embed_weights)
